# Initial kernel scaffold; baseline (speedup 1.0000x reference)
import functools

import jax
import jax.numpy as jnp
from jax import lax
from jax.experimental import pallas as pl
from jax.experimental.pallas import tpu as pltpu
from jax.experimental.pallas import tpu_sc as plsc

_VOCAB = 50
_HIDDEN = 32
_BATCH = 4096
_SEQ = 200
_N = _BATCH * _SEQ

_NC = 2
_NS = 16
_NW = _NC * _NS
_ROWS_PER_W = _N // _NW
_IDXBLK = 128
_BLKS_PER_CHUNK = 8
_CHUNK = _IDXBLK * _BLKS_PER_CHUNK
_N_CHUNKS = _ROWS_PER_W // _CHUNK


def _fuse_table_body(e_ref, w_ref, b_ref, t_ref):
    t_ref[...] = (
        jnp.dot(e_ref[...], w_ref[...], preferred_element_type=jnp.float32)
        + b_ref[...]
    )


def _fuse_table(embed_table, W, b):
    return pl.pallas_call(
        _fuse_table_body,
        out_shape=jax.ShapeDtypeStruct((_VOCAB, _VOCAB), jnp.float32),
    )(embed_table, W, b.reshape(1, _VOCAB))


def _gather_body(t_hbm, idx_hbm, out_hbm, idx_v, rows_v, sem):
    wid = lax.axis_index("s") * _NC + lax.axis_index("c")
    base_blk = wid * (_ROWS_PER_W // _IDXBLK)

    def chunk_body(c, _):
        blk0 = base_blk + c * _BLKS_PER_CHUNK
        row0 = blk0 * _IDXBLK
        pltpu.sync_copy(idx_hbm.at[pl.ds(blk0, _BLKS_PER_CHUNK)], idx_v)
        handles = [
            pltpu.async_copy(
                t_hbm.at[idx_v.at[j]],
                rows_v.at[pl.ds(j * _IDXBLK, _IDXBLK)],
                sem,
            )
            for j in range(_BLKS_PER_CHUNK)
        ]
        for h in handles:
            h.wait()
        pltpu.sync_copy(rows_v, out_hbm.at[pl.ds(row0, _CHUNK)])
        return 0

    lax.fori_loop(0, _N_CHUNKS, chunk_body, 0)


def _sc_gather(table, idx2d):
    mesh = plsc.VectorSubcoreMesh(core_axis_name="c", subcore_axis_name="s")
    kern = functools.partial(
        pl.kernel,
        mesh=mesh,
        out_type=jax.ShapeDtypeStruct((_N, _VOCAB), jnp.float32),
        scratch_types=[
            pltpu.VMEM((_BLKS_PER_CHUNK, _IDXBLK), jnp.int32),
            pltpu.VMEM((_CHUNK, _VOCAB), jnp.float32),
            pltpu.SemaphoreType.DMA,
        ],
    )(_gather_body)
    return kern(table, idx2d)


def kernel(indices, embed_table, W, b):
    table = _fuse_table(embed_table, W, b)
    idx2d = indices.reshape(_N // _IDXBLK, _IDXBLK)
    out = _sc_gather(table, idx2d)
    return out.reshape(_BATCH, _SEQ, _VOCAB)


# speedup vs baseline: 1.6461x; 1.6461x over previous
import functools

import jax
import jax.numpy as jnp
from jax import lax
from jax.experimental import pallas as pl
from jax.experimental.pallas import tpu as pltpu
from jax.experimental.pallas import tpu_sc as plsc

_VOCAB = 50
_HIDDEN = 32
_BATCH = 4096
_SEQ = 200
_N = _BATCH * _SEQ

_TPAD = 64
_NC = 2
_NS = 16
_NW = _NC * _NS
_ROWS_PER_W = _N // _NW
_CHUNK = 1024
_GROUPS = _CHUNK // 16
_N_CHUNKS = _ROWS_PER_W // _CHUNK


def _fuse_table_body(e_ref, w_ref, b_ref, t_ref):
    t_ref[...] = (
        jnp.dot(e_ref[...], w_ref[...], preferred_element_type=jnp.float32)
        + b_ref[...]
    )


def _fuse_table(embed_table, W, b):
    wp = jnp.zeros((_HIDDEN, _TPAD), jnp.float32).at[:, :_VOCAB].set(W)
    bp = jnp.zeros((1, _TPAD), jnp.float32).at[0, :_VOCAB].set(b)
    t = pl.pallas_call(
        _fuse_table_body,
        out_shape=jax.ShapeDtypeStruct((_VOCAB, _TPAD), jnp.float32),
    )(embed_table, wp, bp)
    return t.reshape(_VOCAB * _TPAD)


def _gather_body(t_hbm, idx_hbm, out_hbm, table_v, idx_v, out_v, sem):
    wid = lax.axis_index("s") * _NC + lax.axis_index("c")
    base_row = wid * _ROWS_PER_W

    pltpu.sync_copy(t_hbm, table_v)

    lane = lax.iota(jnp.int32, 16)
    lane50 = lane * _VOCAB

    def chunk_body(c, _):
        row0 = base_row + c * _CHUNK
        pltpu.sync_copy(idx_hbm.at[pl.ds(row0, _CHUNK)], idx_v)

        def group_body(g, _):
            rowids = idx_v[pl.ds(g * 16, 16)]
            fb = rowids * _TPAD
            pos = lane50 + g * (16 * _VOCAB)
            for col in range(_VOCAB):
                vals = plsc.load_gather(table_v, [fb + col])
                plsc.store_scatter(out_v, [pos + col], vals)
            return 0

        lax.fori_loop(0, _GROUPS, group_body, 0)
        pltpu.sync_copy(out_v, out_hbm.at[pl.ds(row0 * _VOCAB, _CHUNK * _VOCAB)])
        return 0

    lax.fori_loop(0, _N_CHUNKS, chunk_body, 0)


def _sc_gather(table_flat, idx_flat):
    mesh = plsc.VectorSubcoreMesh(core_axis_name="c", subcore_axis_name="s")
    kern = functools.partial(
        pl.kernel,
        mesh=mesh,
        compiler_params=pltpu.CompilerParams(needs_layout_passes=False),
        out_type=jax.ShapeDtypeStruct((_N * _VOCAB,), jnp.float32),
        scratch_types=[
            pltpu.VMEM((_VOCAB * _TPAD,), jnp.float32),
            pltpu.VMEM((_CHUNK,), jnp.int32),
            pltpu.VMEM((_CHUNK * _VOCAB,), jnp.float32),
            pltpu.SemaphoreType.DMA,
        ],
    )(_gather_body)
    return kern(table_flat, idx_flat)


def kernel(indices, embed_table, W, b):
    table_flat = _fuse_table(embed_table, W, b)
    out = _sc_gather(table_flat, indices.reshape(_N))
    return out.reshape(_BATCH, _SEQ, _VOCAB)
